# baseline (speedup 1.0000x reference)
import jax
import jax.numpy as jnp
import numpy as np
from jax import lax
from jax.experimental import pallas as pl
from jax.experimental.pallas import tpu as pltpu
from jax.experimental.pallas import tpu_sc as plsc

_BATCH = 32
_NCARD = 2
_NACT = 3
_L = 16

def _gumbel_const():
    g = jax.random.gumbel(jax.random.key(42), (_NCARD * _BATCH, _NACT),
                          jnp.float32)
    return g.reshape(_BATCH, _NCARD, _NACT).transpose(1, 2, 0)

_LN2 = 0.6931471805599453


def _log16(x):
    bits = lax.bitcast_convert_type(x, jnp.int32)
    e = lax.shift_right_logical(bits, 23) - 127
    m = lax.bitcast_convert_type(
        (bits & jnp.int32(0x007FFFFF)) | jnp.int32(0x3F800000), jnp.float32)
    z = (m - 1.0) / (m + 1.0)
    z2 = z * z
    p = 1.0 + z2 * (1.0 / 3.0 + z2 * (0.2 + z2 * (1.0 / 7.0 + z2 * (1.0 / 9.0))))
    return e.astype(jnp.float32) * _LN2 + 2.0 * z * p


def _body(cards_hbm, wb_hbm, g_hbm, u0_hbm, bel_hbm, lcf_hbm,
          cards_v, wb_v, g_v, u0_v, bel_v, lcf_v):
    @pl.when((lax.axis_index("c") == 0) & (lax.axis_index("s") == 0))
    def _():
        pltpu.sync_copy(cards_hbm, cards_v)
        pltpu.sync_copy(wb_hbm, wb_v)
        pltpu.sync_copy(g_hbm, g_v)

        def fulli(v):
            return jnp.full((_L,), v, jnp.int32)

        def fullf(v):
            return jnp.full((_L,), v, jnp.float32)

        wvec = wb_v[...]

        def bcast(i):
            return lax.broadcast(wvec[i], (_L,))

        lvec = [[bcast(2 * a + c) + bcast(6 + a) for a in range(_NACT)]
                for c in range(_NCARD)]
        lse = [_log16(jnp.exp(lvec[c][0]) + jnp.exp(lvec[c][1])
                      + jnp.exp(lvec[c][2]))
               for c in range(_NCARD)]
        iota = lax.iota(jnp.int32, _L)
        for h in range(_BATCH // _L):
            cards_h = cards_v[pl.ds(_L * h, _L)]
            cf, lcf = [], []
            for c in range(_NCARD):
                z = [lvec[c][a] + g_v[c, a, pl.ds(_L * h, _L)]
                     for a in range(_NACT)]
                cf_c = jnp.where(z[0] >= z[1],
                                 jnp.where(z[0] >= z[2], fulli(0), fulli(2)),
                                 jnp.where(z[1] >= z[2], fulli(1), fulli(2)))
                lcf_c = jnp.where(cf_c == 0, lvec[c][0],
                                  jnp.where(cf_c == 1, lvec[c][1],
                                            lvec[c][2])) - lse[c]
                cf.append(cf_c)
                lcf.append(lcf_c)
            u0_h = jnp.where(cards_h == 0, cf[0], cf[1])
            w = [jnp.where(cf[c] == u0_h, fullf(1.0), fullf(0.0))
                 for c in range(_NCARD)]
            s = w[0] + w[1]
            bel = [w[0] / s, w[1] / s]
            u0_v[pl.ds(_L * h, _L)] = u0_h
            half = lax.shift_right_logical(iota, 1)
            parity = iota & 1
            for q in range(2):
                idxq = half + 8 * q
                bel_vec = jnp.where(
                    parity == 0,
                    bel[0].at[idxq].get(mode="promise_in_bounds"),
                    bel[1].at[idxq].get(mode="promise_in_bounds"))
                lcf_vec = jnp.where(
                    parity == 0,
                    lcf[0].at[idxq].get(mode="promise_in_bounds"),
                    lcf[1].at[idxq].get(mode="promise_in_bounds"))
                bel_v[2 * h + q, :] = bel_vec
                lcf_v[2 * h + q, :] = lcf_vec
        pltpu.sync_copy(u0_v, u0_hbm)
        pltpu.sync_copy(bel_v, bel_hbm)
        pltpu.sync_copy(lcf_v, lcf_hbm)


_sc_call = pl.kernel(
    _body,
    out_type=(
        jax.ShapeDtypeStruct((_BATCH,), jnp.int32),
        jax.ShapeDtypeStruct((2 * _BATCH // _L, _L), jnp.float32),
        jax.ShapeDtypeStruct((2 * _BATCH // _L, _L), jnp.float32),
    ),
    mesh=plsc.VectorSubcoreMesh(core_axis_name="c", subcore_axis_name="s"),
    scratch_types=[
        pltpu.VMEM((_BATCH,), jnp.int32),
        pltpu.VMEM((_L,), jnp.float32),
        pltpu.VMEM((_NCARD, _NACT, _BATCH), jnp.float32),
        pltpu.VMEM((_BATCH,), jnp.int32),
        pltpu.VMEM((2 * _BATCH // _L, _L), jnp.float32),
        pltpu.VMEM((2 * _BATCH // _L, _L), jnp.float32),
    ],
    name="a0_sample_sc",
)


def kernel(cards_0, W, b):
    wb = jnp.concatenate([W.reshape(-1), b, jnp.zeros((7,), jnp.float32)])
    u0, bel, lcf = _sc_call(cards_0.astype(jnp.int32), wb, _gumbel_const())
    return (u0, bel.reshape(_BATCH, _NCARD), lcf.reshape(_BATCH, _NCARD))


# speedup vs baseline: 1.0893x; 1.0893x over previous
import jax
import jax.numpy as jnp
from jax import lax
from jax.experimental import pallas as pl
from jax.experimental.pallas import tpu as pltpu
from jax.experimental.pallas import tpu_sc as plsc

_BATCH = 32
_NCARD = 2
_NACT = 3
_L = 16

_IN_WORDS = 16 + _BATCH + _NCARD * _NACT * _BATCH
_G_OFF = 16 + _BATCH
_OUT_WORDS = _BATCH + 2 * _NCARD * _BATCH

_LN2 = 0.6931471805599453


def _gumbel_const():
    g = jax.random.gumbel(jax.random.key(42), (_NCARD * _BATCH, _NACT),
                          jnp.float32)
    return g.reshape(_BATCH, _NCARD, _NACT).transpose(1, 2, 0).reshape(-1)


def _log16(x):
    bits = lax.bitcast_convert_type(x, jnp.int32)
    e = lax.shift_right_logical(bits, 23) - 127
    m = lax.bitcast_convert_type(
        (bits & jnp.int32(0x007FFFFF)) | jnp.int32(0x3F800000), jnp.float32)
    z = (m - 1.0) / (m + 1.0)
    z2 = z * z
    p = 1.0 + z2 * (1.0 / 3.0 + z2 * (0.2 + z2 * (1.0 / 7.0 + z2 * (1.0 / 9.0))))
    return e.astype(jnp.float32) * _LN2 + 2.0 * z * p


def _body(in_hbm, out_hbm, in_v, out_v):
    pltpu.sync_copy(in_hbm, in_v)

    def fulli(v):
        return jnp.full((_L,), v, jnp.int32)

    def fullf(v):
        return jnp.full((_L,), v, jnp.float32)

    wvec = in_v[pl.ds(0, _L)]

    def bcast(i):
        return lax.broadcast(wvec[i], (_L,))

    lvec = [[bcast(2 * a + c) + bcast(6 + a) for a in range(_NACT)]
            for c in range(_NCARD)]
    lse = [_log16(jnp.exp(lvec[c][0]) + jnp.exp(lvec[c][1])
                  + jnp.exp(lvec[c][2]))
           for c in range(_NCARD)]
    iota = lax.iota(jnp.int32, _L)
    half = lax.shift_right_logical(iota, 1)
    parity = iota & 1
    for h in range(_BATCH // _L):
        cards_h = lax.bitcast_convert_type(
            in_v[pl.ds(16 + _L * h, _L)], jnp.int32)
        cf, lcf = [], []
        for c in range(_NCARD):
            z = [lvec[c][a]
                 + in_v[pl.ds(_G_OFF + (c * _NACT + a) * _BATCH + _L * h, _L)]
                 for a in range(_NACT)]
            cf_c = jnp.where(z[0] >= z[1],
                             jnp.where(z[0] >= z[2], fulli(0), fulli(2)),
                             jnp.where(z[1] >= z[2], fulli(1), fulli(2)))
            lcf_c = jnp.where(cf_c == 0, lvec[c][0],
                              jnp.where(cf_c == 1, lvec[c][1],
                                        lvec[c][2])) - lse[c]
            cf.append(cf_c)
            lcf.append(lcf_c)
        u0_h = jnp.where(cards_h == 0, cf[0], cf[1])
        w = [jnp.where(cf[c] == u0_h, fullf(1.0), fullf(0.0))
             for c in range(_NCARD)]
        s = w[0] + w[1]
        bel = [w[0] / s, w[1] / s]
        out_v[pl.ds(_L * h, _L)] = lax.bitcast_convert_type(u0_h, jnp.float32)
        for q in range(2):
            idxq = half + 8 * q
            bel_vec = jnp.where(
                parity == 0,
                bel[0].at[idxq].get(mode="promise_in_bounds"),
                bel[1].at[idxq].get(mode="promise_in_bounds"))
            lcf_vec = jnp.where(
                parity == 0,
                lcf[0].at[idxq].get(mode="promise_in_bounds"),
                lcf[1].at[idxq].get(mode="promise_in_bounds"))
            out_v[pl.ds(_BATCH + _L * (2 * h + q), _L)] = bel_vec
            out_v[pl.ds(3 * _BATCH + _L * (2 * h + q), _L)] = lcf_vec
    pltpu.sync_copy(out_v, out_hbm)


_sc_call = pl.kernel(
    _body,
    out_type=jax.ShapeDtypeStruct((_OUT_WORDS,), jnp.float32),
    mesh=plsc.VectorSubcoreMesh(core_axis_name="c", subcore_axis_name="s",
                                num_cores=1, num_subcores=1),
    scratch_types=[
        pltpu.VMEM((_IN_WORDS,), jnp.float32),
        pltpu.VMEM((_OUT_WORDS,), jnp.float32),
    ],
    name="a0_sample_sc",
)


def kernel(cards_0, W, b):
    packed_in = jnp.concatenate([
        W.reshape(-1), b, jnp.zeros((7,), jnp.float32),
        lax.bitcast_convert_type(cards_0.astype(jnp.int32), jnp.float32),
        _gumbel_const(),
    ])
    out = _sc_call(packed_in)
    u0 = lax.bitcast_convert_type(out[:_BATCH], jnp.int32)
    bel = out[_BATCH:3 * _BATCH].reshape(_BATCH, _NCARD)
    lcf = out[3 * _BATCH:].reshape(_BATCH, _NCARD)
    return (u0, bel, lcf)
